# baseline (speedup 1.0000x reference)
import functools

import jax
import jax.numpy as jnp
from jax import lax
from jax.experimental import pallas as pl
from jax.experimental.pallas import tpu as pltpu
from jax.experimental.pallas import tpu_sc as plsc

VOCAB = 1000000
EMBED_DIM = 32
BATCH = 4096
HIST_LEN = 200
PACK = 128 // EMBED_DIM
VQ = VOCAB // PACK

NUM_CORES = 2
NUM_SUBCORES = 16
NUM_WORKERS = NUM_CORES * NUM_SUBCORES

TOTAL = BATCH * HIST_LEN
TQ = TOTAL // PACK
B_PER_W = TOTAL // NUM_WORKERS
W_PER_BAND = NUM_WORKERS // PACK
CHUNK = 640
N_CHUNKS = B_PER_W // CHUNK
NBUF = 4
AHEAD = 2

DEPAD_BLK = 1000
REPAD_BE = 16


def _depad_body(a_ref, b_ref, c_ref, d_ref, o_ref):
    o_ref[...] = jnp.concatenate(
        [a_ref[...], b_ref[...], c_ref[...], d_ref[...]], axis=-1
    )


_depad = pl.pallas_call(
    _depad_body,
    grid=(VQ // DEPAD_BLK,),
    in_specs=[
        pl.BlockSpec((DEPAD_BLK, EMBED_DIM), lambda i: (i, 0))
        for _ in range(PACK)
    ],
    out_specs=pl.BlockSpec((DEPAD_BLK, 128), lambda i: (i, 0)),
    out_shape=jax.ShapeDtypeStruct((VQ, 128), jnp.float32),
    compiler_params=pltpu.CompilerParams(needs_layout_passes=True),
)


def _repad_body(c_ref, o_ref):
    q = pl.program_id(0)
    x = c_ref[...]
    bands = [x[:, 32 * k:32 * (k + 1)] for k in range(PACK)]
    w = jnp.where(q == 0, bands[0],
                  jnp.where(q == 1, bands[1],
                            jnp.where(q == 2, bands[2], bands[3])))
    o_ref[...] = w.reshape(REPAD_BE, HIST_LEN, EMBED_DIM)


_REPAD_ROWS = REPAD_BE * HIST_LEN
_REPAD_STEPS = TQ // _REPAD_ROWS

_repad = pl.pallas_call(
    _repad_body,
    grid=(PACK, _REPAD_STEPS),
    in_specs=[pl.BlockSpec((_REPAD_ROWS, 128), lambda q, i: (i, 0))],
    out_specs=pl.BlockSpec(
        (REPAD_BE, HIST_LEN, EMBED_DIM),
        lambda q, i: (q * _REPAD_STEPS + i, 0, 0),
    ),
    out_shape=jax.ShapeDtypeStruct((BATCH, HIST_LEN, EMBED_DIM), jnp.float32),
    compiler_params=pltpu.CompilerParams(needs_layout_passes=True),
)


def _make_sc_kernel():
    mesh = plsc.VectorSubcoreMesh(
        core_axis_name="c", subcore_axis_name="s",
        num_cores=NUM_CORES, num_subcores=NUM_SUBCORES,
    )

    @functools.partial(
        pl.kernel,
        out_type=jax.ShapeDtypeStruct((TQ, 128), jnp.float32),
        mesh=mesh,
        scratch_types=[
            pltpu.VMEM((B_PER_W,), jnp.int32),
            pltpu.VMEM((NBUF, CHUNK, EMBED_DIM), jnp.float32),
            [pltpu.SemaphoreType.DMA] * NBUF,
            [pltpu.SemaphoreType.DMA] * NBUF,
        ],
        compiler_params=pltpu.CompilerParams(use_tc_tiling_on_sc=False),
    )
    def emb_kernel(idx_hbm, table_hbm, out_hbm, idx_v, rows_v, gsems, osems):
        wid = lax.axis_index("s") * NUM_CORES + lax.axis_index("c")
        base = wid * B_PER_W
        band = wid // W_PER_BAND
        krow = (wid % W_PER_BAND) * B_PER_W
        pltpu.sync_copy(idx_hbm.at[pl.ds(base, B_PER_W)], idx_v)

        def gcopy(c, b):
            return pltpu.make_async_copy(
                table_hbm.at[idx_v.at[pl.ds(c * CHUNK, CHUNK)]],
                rows_v.at[b], gsems[b],
            )

        def gstart(c, b):
            gcopy(c, b).start()

        def gwait(c, b):
            gcopy(c, b).wait()

        def ocopy(c, b):
            return pltpu.make_async_copy(
                rows_v.at[b],
                out_hbm.at[pl.ds(krow + c * CHUNK, CHUNK),
                           pl.ds(band * EMBED_DIM, EMBED_DIM)],
                osems[b],
            )

        def ostart(c, b):
            ocopy(c, b).start()

        def owait(c, b):
            ocopy(c, b).wait()

        def step(c, b):
            gwait(c, b)
            ostart(c, b)
            b2 = (b + AHEAD) % NBUF
            owait(c - AHEAD, b2)
            gstart(c + AHEAD, b2)

        gstart(0, 0)
        gstart(1, 1)
        gwait(0, 0); ostart(0, 0); gstart(2, 2)
        gwait(1, 1); ostart(1, 1); gstart(3, 3)
        gwait(2, 2); ostart(2, 2); owait(0, 0); gstart(4, 0)
        gwait(3, 3); ostart(3, 3); owait(1, 1); gstart(5, 1)

        def group(g, carry):
            c0 = g * NBUF
            for b in range(NBUF):
                step(c0 + b, b)
            return carry

        lax.fori_loop(1, N_CHUNKS // NBUF - 1, group, 0)

        c = N_CHUNKS - NBUF
        gwait(c + 0, 0); ostart(c + 0, 0); owait(c - 2, 2); gstart(c + 2, 2)
        gwait(c + 1, 1); ostart(c + 1, 1); owait(c - 1, 3); gstart(c + 3, 3)
        gwait(c + 2, 2); ostart(c + 2, 2)
        gwait(c + 3, 3); ostart(c + 3, 3)
        owait(c + 0, 0); owait(c + 1, 1); owait(c + 2, 2); owait(c + 3, 3)

    return emb_kernel


_emb_kernel = _make_sc_kernel()


def kernel(x, embedding_matrix):
    flat_idx = x.reshape(-1).astype(jnp.int32)
    m_idx = PACK * (flat_idx % VQ) + flat_idx // VQ
    quarters = [
        lax.slice_in_dim(embedding_matrix, q * VQ, (q + 1) * VQ)
        for q in range(PACK)
    ]
    tabc = _depad(*quarters)
    tab_lin = tabc.reshape(VOCAB, EMBED_DIM)
    outc = _emb_kernel(m_idx, tab_lin)
    out3 = outc.reshape(TQ, PACK, EMBED_DIM).transpose(1, 0, 2)
    return out3.reshape(BATCH, HIST_LEN, EMBED_DIM)


# speedup vs baseline: 1.3474x; 1.3474x over previous
import functools

import jax
import jax.numpy as jnp
from jax import lax
from jax.experimental import pallas as pl
from jax.experimental.pallas import tpu as pltpu
from jax.experimental.pallas import tpu_sc as plsc

VOCAB = 1000000
EMBED_DIM = 32
BATCH = 4096
HIST_LEN = 200
PACK = 128 // EMBED_DIM
VQ = VOCAB // PACK

NUM_CORES = 2
NUM_SUBCORES = 16
NUM_WORKERS = NUM_CORES * NUM_SUBCORES

TOTAL = BATCH * HIST_LEN
TQ = TOTAL // PACK
B_PER_W = TOTAL // NUM_WORKERS
W_PER_BAND = NUM_WORKERS // PACK
CHUNK = 640
N_CHUNKS = B_PER_W // CHUNK
NBUF = 4
AHEAD = 2

DEPAD_BLK = 1000
REPAD_BE = 16


def _depad_body(a_ref, b_ref, c_ref, d_ref, o_ref):
    o_ref[...] = jnp.concatenate(
        [a_ref[...], b_ref[...], c_ref[...], d_ref[...]], axis=-1
    )


_depad = pl.pallas_call(
    _depad_body,
    grid=(VQ // DEPAD_BLK,),
    in_specs=[
        pl.BlockSpec((DEPAD_BLK, EMBED_DIM), lambda i: (i, 0))
        for _ in range(PACK)
    ],
    out_specs=pl.BlockSpec((DEPAD_BLK, 128), lambda i: (i, 0)),
    out_shape=jax.ShapeDtypeStruct((VQ, 128), jnp.float32),
    compiler_params=pltpu.CompilerParams(needs_layout_passes=True),
)


_REPAD_H = 8
_B1 = BATCH // PACK


def _repad_body(c_ref, o_ref):
    x = c_ref[...]
    for h in range(_REPAD_H):
        for q in range(PACK):
            blk = x[:, h, 32 * q:32 * (q + 1)]
            o_ref[h, :, 1024 * q:1024 * (q + 1)] = blk.T


_repad = pl.pallas_call(
    _repad_body,
    grid=(HIST_LEN // _REPAD_H,),
    in_specs=[pl.BlockSpec((_B1, _REPAD_H, 128), lambda i: (0, i, 0))],
    out_specs=pl.BlockSpec(
        (_REPAD_H, EMBED_DIM, BATCH), lambda i: (i, 0, 0)
    ),
    out_shape=jax.ShapeDtypeStruct((HIST_LEN, EMBED_DIM, BATCH), jnp.float32),
    compiler_params=pltpu.CompilerParams(needs_layout_passes=True),
)


def _make_sc_kernel():
    mesh = plsc.VectorSubcoreMesh(
        core_axis_name="c", subcore_axis_name="s",
        num_cores=NUM_CORES, num_subcores=NUM_SUBCORES,
    )

    @functools.partial(
        pl.kernel,
        out_type=jax.ShapeDtypeStruct((TQ, 128), jnp.float32),
        mesh=mesh,
        scratch_types=[
            pltpu.VMEM((B_PER_W,), jnp.int32),
            pltpu.VMEM((NBUF, CHUNK, EMBED_DIM), jnp.float32),
            [pltpu.SemaphoreType.DMA] * NBUF,
            [pltpu.SemaphoreType.DMA] * NBUF,
        ],
        compiler_params=pltpu.CompilerParams(use_tc_tiling_on_sc=False),
    )
    def emb_kernel(idx_hbm, table_hbm, out_hbm, idx_v, rows_v, gsems, osems):
        wid = lax.axis_index("s") * NUM_CORES + lax.axis_index("c")
        base = wid * B_PER_W
        band = wid // W_PER_BAND
        krow = (wid % W_PER_BAND) * B_PER_W
        pltpu.sync_copy(idx_hbm.at[pl.ds(base, B_PER_W)], idx_v)

        def gcopy(c, b):
            return pltpu.make_async_copy(
                table_hbm.at[idx_v.at[pl.ds(c * CHUNK, CHUNK)]],
                rows_v.at[b], gsems[b],
            )

        def gstart(c, b):
            gcopy(c, b).start()

        def gwait(c, b):
            gcopy(c, b).wait()

        def ocopy(c, b):
            return pltpu.make_async_copy(
                rows_v.at[b],
                out_hbm.at[pl.ds(krow + c * CHUNK, CHUNK),
                           pl.ds(band * EMBED_DIM, EMBED_DIM)],
                osems[b],
            )

        def ostart(c, b):
            ocopy(c, b).start()

        def owait(c, b):
            ocopy(c, b).wait()

        def step(c, b):
            gwait(c, b)
            ostart(c, b)
            b2 = (b + AHEAD) % NBUF
            owait(c - AHEAD, b2)
            gstart(c + AHEAD, b2)

        gstart(0, 0)
        gstart(1, 1)
        gwait(0, 0); ostart(0, 0); gstart(2, 2)
        gwait(1, 1); ostart(1, 1); gstart(3, 3)
        gwait(2, 2); ostart(2, 2); owait(0, 0); gstart(4, 0)
        gwait(3, 3); ostart(3, 3); owait(1, 1); gstart(5, 1)

        def group(g, carry):
            c0 = g * NBUF
            for b in range(NBUF):
                step(c0 + b, b)
            return carry

        lax.fori_loop(1, N_CHUNKS // NBUF - 1, group, 0)

        c = N_CHUNKS - NBUF
        gwait(c + 0, 0); ostart(c + 0, 0); owait(c - 2, 2); gstart(c + 2, 2)
        gwait(c + 1, 1); ostart(c + 1, 1); owait(c - 1, 3); gstart(c + 3, 3)
        gwait(c + 2, 2); ostart(c + 2, 2)
        gwait(c + 3, 3); ostart(c + 3, 3)
        owait(c + 0, 0); owait(c + 1, 1); owait(c + 2, 2); owait(c + 3, 3)

    return emb_kernel


_emb_kernel = _make_sc_kernel()


def kernel(x, embedding_matrix):
    flat_idx = x.reshape(-1).astype(jnp.int32)
    m_idx = PACK * (flat_idx % VQ) + flat_idx // VQ
    quarters = [
        lax.slice_in_dim(embedding_matrix, q * VQ, (q + 1) * VQ)
        for q in range(PACK)
    ]
    tabc = _depad(*quarters)
    tab_lin = tabc.reshape(VOCAB, EMBED_DIM)
    outc = _emb_kernel(m_idx, tab_lin)
    inter3 = outc.reshape(_B1, HIST_LEN, 128)
    out_t = _repad(inter3)
    return jnp.transpose(out_t, (2, 0, 1))


# speedup vs baseline: 1.3694x; 1.0164x over previous
import functools

import jax
import jax.numpy as jnp
from jax import lax
from jax.experimental import pallas as pl
from jax.experimental.pallas import tpu as pltpu
from jax.experimental.pallas import tpu_sc as plsc

VOCAB = 1000000
EMBED_DIM = 32
BATCH = 4096
HIST_LEN = 200

NUM_CORES = 2
NUM_SUBCORES = 16
NUM_WORKERS = NUM_CORES * NUM_SUBCORES

TOTAL = BATCH * HIST_LEN
B_PER_W = TOTAL // NUM_WORKERS
CHUNK = 640
N_CHUNKS = B_PER_W // CHUNK
NBUF = 4
AHEAD = 2


def _make_kernel():
    mesh = plsc.VectorSubcoreMesh(
        core_axis_name="c", subcore_axis_name="s",
        num_cores=NUM_CORES, num_subcores=NUM_SUBCORES,
    )

    @functools.partial(
        pl.kernel,
        out_type=jax.ShapeDtypeStruct((TOTAL, EMBED_DIM), jnp.float32),
        mesh=mesh,
        scratch_types=[
            pltpu.VMEM((B_PER_W,), jnp.int32),
            pltpu.VMEM((NBUF, CHUNK, EMBED_DIM), jnp.float32),
            [pltpu.SemaphoreType.DMA] * NBUF,
            [pltpu.SemaphoreType.DMA] * NBUF,
        ],
        compiler_params=pltpu.CompilerParams(use_tc_tiling_on_sc=False),
    )
    def emb_kernel(idx_hbm, table_hbm, out_hbm, idx_v, rows_v, gsems, osems):
        wid = lax.axis_index("s") * NUM_CORES + lax.axis_index("c")
        base = wid * B_PER_W
        pltpu.sync_copy(idx_hbm.at[pl.ds(base, B_PER_W)], idx_v)

        def gcopy(c, b):
            return pltpu.make_async_copy(
                table_hbm.at[idx_v.at[pl.ds(c * CHUNK, CHUNK)]],
                rows_v.at[b], gsems[b],
            )

        def gstart(c, b):
            gcopy(c, b).start()

        def gwait(c, b):
            gcopy(c, b).wait()

        def ocopy(c, b):
            return pltpu.make_async_copy(
                rows_v.at[b],
                out_hbm.at[pl.ds(base + c * CHUNK, CHUNK)], osems[b],
            )

        def ostart(c, b):
            ocopy(c, b).start()

        def owait(c, b):
            ocopy(c, b).wait()

        def step(c, b):
            gwait(c, b)
            ostart(c, b)
            b2 = (b + AHEAD) % NBUF
            owait(c - AHEAD, b2)
            gstart(c + AHEAD, b2)

        gstart(0, 0)
        gstart(1, 1)
        gwait(0, 0); ostart(0, 0); gstart(2, 2)
        gwait(1, 1); ostart(1, 1); gstart(3, 3)
        gwait(2, 2); ostart(2, 2); owait(0, 0); gstart(4, 0)
        gwait(3, 3); ostart(3, 3); owait(1, 1); gstart(5, 1)

        def group(g, carry):
            c0 = g * NBUF
            for b in range(NBUF):
                step(c0 + b, b)
            return carry

        lax.fori_loop(1, N_CHUNKS // NBUF - 1, group, 0)

        c = N_CHUNKS - NBUF
        gwait(c + 0, 0); ostart(c + 0, 0); owait(c - 2, 2); gstart(c + 2, 2)
        gwait(c + 1, 1); ostart(c + 1, 1); owait(c - 1, 3); gstart(c + 3, 3)
        gwait(c + 2, 2); ostart(c + 2, 2)
        gwait(c + 3, 3); ostart(c + 3, 3)
        owait(c + 0, 0); owait(c + 1, 1); owait(c + 2, 2); owait(c + 3, 3)

    return emb_kernel


_emb_kernel = _make_kernel()


def kernel(x, embedding_matrix):
    flat_idx = x.reshape(-1).astype(jnp.int32)
    out = _emb_kernel(flat_idx, embedding_matrix)
    return out.reshape(BATCH, HIST_LEN, EMBED_DIM)
